# Initial kernel scaffold; baseline (speedup 1.0000x reference)
import functools

import jax
import jax.numpy as jnp
from jax import lax
from jax.experimental import pallas as pl
from jax.experimental.pallas import tpu as pltpu
from jax.experimental.pallas import tpu_sc as plsc

D = 64
B = 16384 * 50
NC = 2
NS = 16
NW = NC * NS
BPW = B // NW
CH = 128
NCH = BPW // CH
NBUF = 2

_mesh = plsc.VectorSubcoreMesh(core_axis_name="c", subcore_axis_name="s")


@functools.partial(
    pl.kernel,
    mesh=_mesh,
    out_type=jax.ShapeDtypeStruct((B, D), jnp.float32),
    scratch_types=[
        pltpu.VMEM((NCH, CH), jnp.int32),
        pltpu.VMEM((CH, D), jnp.float32),
        pltpu.VMEM((CH, D), jnp.float32),
        pltpu.SemaphoreType.DMA,
        pltpu.SemaphoreType.DMA,
    ],
)
def _gather_kernel(idx_hbm, table_hbm, out_hbm, idx_v, rows0, rows1, sem0, sem1):
    wid = lax.axis_index("s") * NC + lax.axis_index("c")
    base = wid * BPW
    pltpu.sync_copy(idx_hbm.at[wid], idx_v)
    rows = (rows0, rows1)
    sems = (sem0, sem1)

    def start(j, b):
        pltpu.async_copy(table_hbm.at[idx_v.at[j]], rows[b], sems[b])

    def wait(b):
        pltpu.make_async_copy(table_hbm.at[idx_v.at[0]], rows[b], sems[b]).wait()

    start(0, 0)
    start(1, 1)

    def body(i, carry):
        for b in range(NBUF):
            j = i * NBUF + b
            wait(b)
            pltpu.sync_copy(rows[b], out_hbm.at[pl.ds(base + j * CH, CH)])
            nj = j + NBUF

            @pl.when(nj < NCH)
            def _():
                start(nj, b)

        return carry

    lax.fori_loop(0, NCH // NBUF, body, 0)


def kernel(x, weight):
    idx = x.reshape(NW, NCH, CH)
    out = _gather_kernel(idx, weight)
    return out.reshape(x.shape[0], x.shape[1], D)


# speedup vs baseline: 1.8382x; 1.8382x over previous
import functools

import jax
import jax.numpy as jnp
from jax import lax
from jax.experimental import pallas as pl
from jax.experimental.pallas import tpu as pltpu
from jax.experimental.pallas import tpu_sc as plsc

D = 64
B = 16384 * 50
NC = 2
NS = 16
NW = NC * NS
BPW = B // NW
CH = 128
NCH = BPW // CH
NBUF = 2

_mesh = plsc.VectorSubcoreMesh(core_axis_name="c", subcore_axis_name="s")


def _gather_body(idx_hbm, table_hbm, out_hbm, idx_v, rows0, rows1, sem0, sem1):
    wid = lax.axis_index("s") * NC + lax.axis_index("c")
    base = wid * BPW
    pltpu.sync_copy(idx_hbm.at[wid], idx_v)
    rows = (rows0, rows1)
    sems = (sem0, sem1)

    def start(j, b):
        pltpu.async_copy(table_hbm.at[idx_v.at[j]], rows[b], sems[b])

    def wait(b):
        pltpu.make_async_copy(table_hbm.at[idx_v.at[0]], rows[b], sems[b]).wait()

    start(0, 0)
    start(1, 1)

    def body(i, carry):
        for b in range(NBUF):
            j = i * NBUF + b
            wait(b)
            pltpu.sync_copy(rows[b], out_hbm.at[pl.ds(base + j * CH, CH)])
            nj = j + NBUF

            @pl.when(nj < NCH)
            def _():
                start(nj, b)

        return carry

    lax.fori_loop(0, NCH // NBUF, body, 0)


_SCRATCH = [
    pltpu.VMEM((NCH, CH), jnp.int32),
    pltpu.VMEM((CH, D), jnp.float32),
    pltpu.VMEM((CH, D), jnp.float32),
    pltpu.SemaphoreType.DMA,
    pltpu.SemaphoreType.DMA,
]

_gather_kernel = pl.kernel(
    _gather_body,
    mesh=_mesh,
    compiler_params=pltpu.CompilerParams(use_tc_tiling_on_sc=False),
    out_type=jax.ShapeDtypeStruct((B, D), jnp.float32),
    scratch_types=_SCRATCH,
)


def kernel(x, weight):
    idx = x.reshape(NW, NCH, CH)
    out = _gather_kernel(idx, weight)
    return out.reshape(x.shape[0], x.shape[1], D)
